# baseline (speedup 1.0000x reference)
import jax
import jax.numpy as jnp
from jax import lax
from jax.experimental import pallas as pl
from jax.experimental.pallas import tpu as pltpu
from jax.experimental.pallas import tpu_sc as plsc

_B = 128
_V = 100000
_NC = 2
_NS = 16
_NW = _NC * _NS
_RPW = _B // _NW
_L = 16
_NVEC = _V // _L
_CHUNK = 20000
_NCHUNK = _V // _CHUNK
_CVEC = _CHUNK // _L
_U = 10


def _sc_body(logits_hbm, scale_hbm, noise_hbm, out_hbm, row_v, g_v, s_v):
    wid = lax.axis_index("s") * _NC + lax.axis_index("c")
    pltpu.sync_copy(scale_hbm, s_v)
    scale = s_v[...]

    for r in range(_RPW):
        base = pl.multiple_of((wid * _RPW + r) * _V, 8)
        pltpu.sync_copy(logits_hbm.at[pl.ds(base, _V)], row_v)

        def p1_chunk(c, m16):
            coff = c * _CHUNK
            pltpu.sync_copy(noise_hbm.at[pl.ds(base + coff, _CHUNK)], g_v)

            def p1_vec(i, m16):
                off = coff + i * (_L * _U)
                goff = i * (_L * _U)
                for u in range(_U):
                    y = (row_v[pl.ds(off + u * _L, _L)] * scale
                         + g_v[pl.ds(goff + u * _L, _L)])
                    row_v[pl.ds(off + u * _L, _L)] = y
                    m16 = jnp.maximum(m16, y)
                return m16

            return lax.fori_loop(0, _CVEC // _U, p1_vec, m16)

        m16 = lax.fori_loop(0, _NCHUNK, p1_chunk,
                            jnp.full((_L,), -jnp.inf, jnp.float32))
        m = m16[0]
        for i in range(1, _L):
            m = jnp.maximum(m, m16[i])

        def p2_vec(i, s16):
            off = i * (_L * _U)
            for u in range(_U):
                e = jnp.exp(row_v[pl.ds(off + u * _L, _L)] - m)
                row_v[pl.ds(off + u * _L, _L)] = e
                s16 = s16 + e
            return s16

        s16 = lax.fori_loop(0, _NVEC // _U, p2_vec,
                            jnp.zeros((_L,), jnp.float32))
        s = s16[0]
        for i in range(1, _L):
            s = s + s16[i]
        r_inv = jnp.ones((_L,), jnp.float32) / (jnp.zeros((_L,), jnp.float32) + s)

        def p3_vec(i, carry):
            off = i * (_L * _U)
            for u in range(_U):
                row_v[pl.ds(off + u * _L, _L)] = (
                    row_v[pl.ds(off + u * _L, _L)] * r_inv)
            return carry

        lax.fori_loop(0, _NVEC // _U, p3_vec, 0)
        pltpu.sync_copy(row_v, out_hbm.at[pl.ds(base, _V)])


_sc_softmax = pl.kernel(
    _sc_body,
    out_type=jax.ShapeDtypeStruct((_B * _V,), jnp.float32),
    mesh=plsc.VectorSubcoreMesh(core_axis_name="c", subcore_axis_name="s",
                                num_cores=_NC, num_subcores=_NS),
    scratch_types=[
        pltpu.VMEM((_V,), jnp.float32),
        pltpu.VMEM((_CHUNK,), jnp.float32),
        pltpu.VMEM((_L,), jnp.float32),
    ],
)


def kernel(logits, temperature, gumbel_noise):
    scale16 = jnp.broadcast_to(jnp.exp(temperature), (_L,)).astype(jnp.float32)
    out = _sc_softmax(logits.reshape(-1), scale16, gumbel_noise.reshape(-1))
    return out.reshape(_B, _V)
